# baseline (speedup 1.0000x reference)
import functools

import jax
import jax.numpy as jnp
from jax import lax
from jax.experimental import pallas as pl
from jax.experimental.pallas import tpu as pltpu
from jax.experimental.pallas import tpu_sc as plsc

NU = 100000
B = 16384
D = 64
L = 16
NC = 2
NS = 16
NW = NC * NS
BPW = B // NW
CHUNK = 128
NCHUNK = BPW // CHUNK
NACC = 4

_mesh = plsc.VectorSubcoreMesh(core_axis_name="c", subcore_axis_name="s")


@functools.partial(
    pl.kernel,
    out_type=jax.ShapeDtypeStruct((B,), jnp.float32),
    mesh=_mesh,
    compiler_params=pltpu.CompilerParams(
        needs_layout_passes=False, use_tc_tiling_on_sc=False),
    scratch_types=[
        pltpu.VMEM((BPW,), jnp.int32),
        pltpu.VMEM((BPW,), jnp.int32),
        pltpu.VMEM((2, CHUNK, D), jnp.float32),
        pltpu.VMEM((2, CHUNK, D), jnp.float32),
        pltpu.VMEM((BPW,), jnp.float32),
        pltpu.VMEM((D * L + L,), jnp.float32),
        pltpu.SemaphoreType.DMA,
        pltpu.SemaphoreType.DMA,
        pltpu.SemaphoreType.DMA,
        pltpu.SemaphoreType.DMA,
    ],
)
def _gmf_sc(uidx_hbm, vidx_hbm, ut_hbm, it_hbm, wb_hbm,
            out_hbm, uidx_v, vidx_v, urows, vrows, outv,
            wv, su0, su1, sv0, sv1):
    wid = lax.axis_index("s") * NC + lax.axis_index("c")
    base = wid * BPW

    pltpu.sync_copy(uidx_hbm.at[pl.ds(base, BPW)], uidx_v)
    pltpu.sync_copy(vidx_hbm.at[pl.ds(base, BPW)], vidx_v)
    pltpu.sync_copy(wb_hbm, wv)

    bvec = wv[pl.ds(D * L, L)]
    lane = lax.iota(jnp.int32, L)
    sems_u = [su0, su1]
    sems_v = [sv0, sv1]

    def start(j):
        bsel = j % 2
        descs = []
        for k in range(CHUNK // L):
            iu = uidx_v[pl.ds(j * CHUNK + k * L, L)]
            iv = vidx_v[pl.ds(j * CHUNK + k * L, L)]
            descs.append(pltpu.async_copy(
                ut_hbm.at[iu], urows.at[bsel, pl.ds(k * L, L)],
                sems_u[bsel]))
            descs.append(pltpu.async_copy(
                it_hbm.at[iv], vrows.at[bsel, pl.ds(k * L, L)],
                sems_v[bsel]))
        return descs

    pend = start(0)
    for j in range(NCHUNK):
        bsel = j % 2
        descs = pend
        if j + 1 < NCHUNK:
            pend = start(j + 1)
        for dsc in descs:
            dsc.wait()
        ub = urows.at[bsel]
        vb = vrows.at[bsel]

        def group_body(g, carry, ub=ub, vb=vb, j=j):
            row = g * L + lane
            accs = []
            for d in range(NACC):
                col = jnp.full((L,), d, dtype=jnp.int32)
                accs.append(plsc.load_gather(ub, [row, col])
                            * plsc.load_gather(vb, [row, col])
                            * wv[pl.ds(d * L, L)])
            for d in range(NACC, D):
                col = jnp.full((L,), d, dtype=jnp.int32)
                accs[d % NACC] += (plsc.load_gather(ub, [row, col])
                                   * plsc.load_gather(vb, [row, col])
                                   * wv[pl.ds(d * L, L)])
            acc = (accs[0] + accs[1]) + (accs[2] + accs[3]) + bvec
            outv[pl.ds(j * CHUNK + g * L, L)] = 1.0 / (1.0 + jnp.exp(-acc))
            return carry

        lax.fori_loop(0, CHUNK // L, group_body, 0)

    pltpu.sync_copy(outv, out_hbm.at[pl.ds(base, BPW)])


def kernel(input, user_table, item_table, W, b):
    idx = input.astype(jnp.int32)
    wb = jnp.concatenate([
        jnp.broadcast_to(W.reshape(D, 1), (D, L)).reshape(D * L),
        jnp.broadcast_to(b, (L,)),
    ])
    return _gmf_sc(idx[:, 0], idx[:, 1], user_table, item_table, wb)


# speedup vs baseline: 1.0304x; 1.0304x over previous
import functools

import jax
import jax.numpy as jnp
from jax import lax
from jax.experimental import pallas as pl
from jax.experimental.pallas import tpu as pltpu
from jax.experimental.pallas import tpu_sc as plsc

NU = 100000
B = 16384
D = 64
L = 16
NC = 2
NS = 16
NW = NC * NS
BPW = B // NW
NG = BPW // L
NBUF = 8
AHEAD = 6
NACC = 4

_mesh = plsc.VectorSubcoreMesh(core_axis_name="c", subcore_axis_name="s")


@functools.partial(
    pl.kernel,
    out_type=jax.ShapeDtypeStruct((B,), jnp.float32),
    mesh=_mesh,
    compiler_params=pltpu.CompilerParams(
        needs_layout_passes=False, use_tc_tiling_on_sc=False),
    scratch_types=[
        pltpu.VMEM((BPW,), jnp.int32),
        pltpu.VMEM((BPW,), jnp.int32),
        pltpu.VMEM((NBUF, L, D), jnp.float32),
        pltpu.VMEM((NBUF, L, D), jnp.float32),
        pltpu.VMEM((BPW,), jnp.float32),
        pltpu.VMEM((D * L + L,), jnp.float32),
        pltpu.SemaphoreType.DMA((NBUF,)),
        pltpu.SemaphoreType.DMA((NBUF,)),
    ],
)
def _gmf_sc(uidx_hbm, vidx_hbm, ut_hbm, it_hbm, wb_hbm,
            out_hbm, uidx_v, vidx_v, urows, vrows, outv,
            wv, sem_u, sem_v):
    wid = lax.axis_index("s") * NC + lax.axis_index("c")
    base = wid * BPW

    pltpu.sync_copy(uidx_hbm.at[pl.ds(base, BPW)], uidx_v)
    pltpu.sync_copy(vidx_hbm.at[pl.ds(base, BPW)], vidx_v)
    pltpu.sync_copy(wb_hbm, wv)

    bvec = wv[pl.ds(D * L, L)]
    lane = lax.iota(jnp.int32, L)

    def issue(g, slot):
        iu = uidx_v[pl.ds(g * L, L)]
        iv = vidx_v[pl.ds(g * L, L)]
        pltpu.async_copy(ut_hbm.at[iu], urows.at[slot], sem_u.at[slot])
        pltpu.async_copy(it_hbm.at[iv], vrows.at[slot], sem_v.at[slot])

    for g in range(AHEAD):
        issue(g, g % NBUF)

    def group_body(g, carry):
        slot = lax.rem(g, NBUF)

        @pl.when(g + AHEAD < NG)
        def _():
            issue(g + AHEAD, lax.rem(g + AHEAD, NBUF))

        pltpu.make_async_copy(ut_hbm.at[uidx_v[pl.ds(g * L, L)]],
                              urows.at[slot], sem_u.at[slot]).wait()
        pltpu.make_async_copy(it_hbm.at[vidx_v[pl.ds(g * L, L)]],
                              vrows.at[slot], sem_v.at[slot]).wait()

        slotv = jnp.full((L,), slot, dtype=jnp.int32)
        accs = []
        for d in range(NACC):
            col = jnp.full((L,), d, dtype=jnp.int32)
            accs.append(plsc.load_gather(urows, [slotv, lane, col])
                        * plsc.load_gather(vrows, [slotv, lane, col])
                        * wv[pl.ds(d * L, L)])
        for d in range(NACC, D):
            col = jnp.full((L,), d, dtype=jnp.int32)
            accs[d % NACC] += (plsc.load_gather(urows, [slotv, lane, col])
                               * plsc.load_gather(vrows, [slotv, lane, col])
                               * wv[pl.ds(d * L, L)])
        acc = (accs[0] + accs[1]) + (accs[2] + accs[3]) + bvec
        outv[pl.ds(g * L, L)] = 1.0 / (1.0 + jnp.exp(-acc))
        return carry

    lax.fori_loop(0, NG, group_body, 0)

    pltpu.sync_copy(outv, out_hbm.at[pl.ds(base, BPW)])


def kernel(input, user_table, item_table, W, b):
    idx = input.astype(jnp.int32)
    wb = jnp.concatenate([
        jnp.broadcast_to(W.reshape(D, 1), (D, L)).reshape(D * L),
        jnp.broadcast_to(b, (L,)),
    ])
    return _gmf_sc(idx[:, 0], idx[:, 1], user_table, item_table, wb)


# speedup vs baseline: 1.0308x; 1.0004x over previous
import functools

import jax
import jax.numpy as jnp
from jax import lax
from jax.experimental import pallas as pl
from jax.experimental.pallas import tpu as pltpu
from jax.experimental.pallas import tpu_sc as plsc

NU = 100000
B = 16384
D = 64
L = 16
NC = 2
NS = 16
NW = NC * NS
BPW = B // NW
NG = BPW // L
NBUF = 8
AHEAD = 1
NACC = 4

_mesh = plsc.VectorSubcoreMesh(core_axis_name="c", subcore_axis_name="s")


@functools.partial(
    pl.kernel,
    out_type=jax.ShapeDtypeStruct((B,), jnp.float32),
    mesh=_mesh,
    compiler_params=pltpu.CompilerParams(
        needs_layout_passes=False, use_tc_tiling_on_sc=False),
    scratch_types=[
        pltpu.VMEM((BPW,), jnp.int32),
        pltpu.VMEM((BPW,), jnp.int32),
        pltpu.VMEM((NBUF, L, D), jnp.float32),
        pltpu.VMEM((NBUF, L, D), jnp.float32),
        pltpu.VMEM((BPW,), jnp.float32),
        pltpu.VMEM((D * L + L,), jnp.float32),
        pltpu.SemaphoreType.DMA((NBUF,)),
        pltpu.SemaphoreType.DMA((NBUF,)),
    ],
)
def _gmf_sc(uidx_hbm, vidx_hbm, ut_hbm, it_hbm, wb_hbm,
            out_hbm, uidx_v, vidx_v, urows, vrows, outv,
            wv, sem_u, sem_v):
    wid = lax.axis_index("s") * NC + lax.axis_index("c")
    base = wid * BPW

    pltpu.sync_copy(uidx_hbm.at[pl.ds(base, BPW)], uidx_v)
    pltpu.sync_copy(vidx_hbm.at[pl.ds(base, BPW)], vidx_v)
    pltpu.sync_copy(wb_hbm, wv)

    bvec = wv[pl.ds(D * L, L)]
    lane = lax.iota(jnp.int32, L)

    def issue(g, slot):
        iu = uidx_v[pl.ds(g * L, L)]
        iv = vidx_v[pl.ds(g * L, L)]
        pltpu.async_copy(ut_hbm.at[iu], urows.at[slot], sem_u.at[slot])
        pltpu.async_copy(it_hbm.at[iv], vrows.at[slot], sem_v.at[slot])

    for g in range(AHEAD):
        issue(g, g % NBUF)

    def group_body(g, carry):
        slot = lax.rem(g, NBUF)

        pltpu.make_async_copy(ut_hbm.at[uidx_v[pl.ds(g * L, L)]],
                              urows.at[slot], sem_u.at[slot]).wait()
        pltpu.make_async_copy(it_hbm.at[vidx_v[pl.ds(g * L, L)]],
                              vrows.at[slot], sem_v.at[slot]).wait()

        @pl.when(g + AHEAD < NG)
        def _():
            issue(g + AHEAD, lax.rem(g + AHEAD, NBUF))

        slotv = jnp.full((L,), slot, dtype=jnp.int32)
        accs = []
        for d in range(NACC):
            col = jnp.full((L,), d, dtype=jnp.int32)
            accs.append(plsc.load_gather(urows, [slotv, lane, col])
                        * plsc.load_gather(vrows, [slotv, lane, col])
                        * wv[pl.ds(d * L, L)])
        for d in range(NACC, D):
            col = jnp.full((L,), d, dtype=jnp.int32)
            accs[d % NACC] += (plsc.load_gather(urows, [slotv, lane, col])
                               * plsc.load_gather(vrows, [slotv, lane, col])
                               * wv[pl.ds(d * L, L)])
        acc = (accs[0] + accs[1]) + (accs[2] + accs[3]) + bvec
        outv[pl.ds(g * L, L)] = 1.0 / (1.0 + jnp.exp(-acc))
        return carry

    lax.fori_loop(0, NG, group_body, 0)

    pltpu.sync_copy(outv, out_hbm.at[pl.ds(base, BPW)])


def kernel(input, user_table, item_table, W, b):
    idx = input.astype(jnp.int32)
    wb = jnp.concatenate([
        jnp.broadcast_to(W.reshape(D, 1), (D, L)).reshape(D * L),
        jnp.broadcast_to(b, (L,)),
    ])
    return _gmf_sc(idx[:, 0], idx[:, 1], user_table, item_table, wb)


# speedup vs baseline: 1.2189x; 1.1825x over previous
import functools

import jax
import jax.numpy as jnp
from jax import lax
from jax.experimental import pallas as pl
from jax.experimental.pallas import tpu as pltpu
from jax.experimental.pallas import tpu_sc as plsc

B = 16384
D = 64
L = 16
NC = 2
NS = 16
NW = NC * NS
BPW = B // NW
CHUNK = 128
NCHUNK = BPW // CHUNK

_mesh = plsc.VectorSubcoreMesh(core_axis_name="c", subcore_axis_name="s")


@functools.partial(
    pl.kernel,
    out_type=jax.ShapeDtypeStruct((B,), jnp.float32),
    mesh=_mesh,
    compiler_params=pltpu.CompilerParams(
        needs_layout_passes=False, use_tc_tiling_on_sc=False),
    scratch_types=[
        pltpu.VMEM((NCHUNK, CHUNK), jnp.int32),
        pltpu.VMEM((NCHUNK, CHUNK), jnp.int32),
        pltpu.VMEM((CHUNK, D), jnp.float32),
        pltpu.VMEM((CHUNK, D), jnp.float32),
        pltpu.VMEM((BPW,), jnp.float32),
        pltpu.VMEM((L * L,), jnp.float32),
        pltpu.VMEM((D + L,), jnp.float32),
        pltpu.SemaphoreType.DMA,
        pltpu.SemaphoreType.DMA,
    ],
)
def _gmf_sc(uidx_hbm, vidx_hbm, ut_hbm, it_hbm, wb_hbm, out_hbm,
            uidx_v, vidx_v, urows, vrows, outv, mat, wv, sem_u, sem_v):
    wid = lax.axis_index("s") * NC + lax.axis_index("c")
    base = wid * BPW

    pltpu.sync_copy(uidx_hbm.at[wid], uidx_v)
    pltpu.sync_copy(vidx_hbm.at[wid], vidx_v)
    pltpu.sync_copy(wb_hbm, wv)

    w = [wv[pl.ds(c * L, L)] for c in range(D // L)]
    bvec = wv[pl.ds(D, L)]
    col_base = lax.iota(jnp.int32, L) * L
    idxcol = [col_base + l for l in range(L)]

    for j in range(NCHUNK):
        cu = pltpu.async_copy(ut_hbm.at[uidx_v.at[j]], urows, sem_u)
        cv = pltpu.async_copy(it_hbm.at[vidx_v.at[j]], vrows, sem_v)
        cu.wait()
        cv.wait()

        def group_body(g, carry, j=j):
            i0 = g * L
            for r in range(L):
                acc = (urows[i0 + r, pl.ds(0, L)]
                       * vrows[i0 + r, pl.ds(0, L)]) * w[0]
                for c in range(1, D // L):
                    acc += (urows[i0 + r, pl.ds(c * L, L)]
                            * vrows[i0 + r, pl.ds(c * L, L)]) * w[c]
                mat[pl.ds(r * L, L)] = acc
            cols = [plsc.load_gather(mat, [idxcol[l]]) for l in range(L)]
            while len(cols) > 1:
                cols = [cols[i] + cols[i + 1] for i in range(0, len(cols), 2)]
            colsum = cols[0] + bvec
            outv[pl.ds(j * CHUNK + i0, L)] = 1.0 / (1.0 + jnp.exp(-colsum))
            return carry

        lax.fori_loop(0, CHUNK // L, group_body, 0)

    pltpu.sync_copy(outv, out_hbm.at[pl.ds(base, BPW)])


def kernel(input, user_table, item_table, W, b):
    idx = input.astype(jnp.int32)
    uidx = idx[:, 0].reshape(NW, NCHUNK, CHUNK)
    vidx = idx[:, 1].reshape(NW, NCHUNK, CHUNK)
    wb = jnp.concatenate([W.reshape(D), jnp.broadcast_to(b, (L,))])
    return _gmf_sc(uidx, vidx, user_table, item_table, wb)
